# baseline (speedup 1.0000x reference)
import functools

import jax
import jax.numpy as jnp
from jax import lax
from jax.experimental import pallas as pl
from jax.experimental.pallas import tpu as pltpu
from jax.experimental.pallas import tpu_sc as plsc

BATCH = 16384
DIM = 32
EPS = 1e-5

@functools.cache
def _make_sc_gather():
  info = plsc.get_sparse_core_info()
  nc, ns = info.num_cores, info.num_subcores
  nw = nc * ns
  b_per_w = BATCH // nw

  def _sc_gather_body(u_idx_hbm, i_idx_hbm, u_table_hbm, i_table_hbm,
                      u_out_hbm, i_out_hbm,
                      idx_u, idx_i, rows_u, rows_i, sem_u, sem_i):
    wid = lax.axis_index("s") * nc + lax.axis_index("c")
    base = wid * b_per_w
    pltpu.sync_copy(u_idx_hbm.at[pl.ds(base, b_per_w)], idx_u)
    pltpu.sync_copy(i_idx_hbm.at[pl.ds(base, b_per_w)], idx_i)
    cp_u = pltpu.async_copy(u_table_hbm.at[idx_u], rows_u, sem_u)
    cp_i = pltpu.async_copy(i_table_hbm.at[idx_i], rows_i, sem_i)
    cp_u.wait()
    pltpu.sync_copy(rows_u, u_out_hbm.at[pl.ds(base, b_per_w)])
    cp_i.wait()
    pltpu.sync_copy(rows_i, i_out_hbm.at[pl.ds(base, b_per_w)])

  return pl.kernel(
      _sc_gather_body,
      mesh=plsc.VectorSubcoreMesh(core_axis_name="c", subcore_axis_name="s"),
      compiler_params=pltpu.CompilerParams(use_tc_tiling_on_sc=False),
      out_type=[
          jax.ShapeDtypeStruct((BATCH, DIM), jnp.float32),
          jax.ShapeDtypeStruct((BATCH, DIM), jnp.float32),
      ],
      scratch_types=[
          pltpu.VMEM((b_per_w,), jnp.int32),
          pltpu.VMEM((b_per_w,), jnp.int32),
          pltpu.VMEM((b_per_w, DIM), jnp.float32),
          pltpu.VMEM((b_per_w, DIM), jnp.float32),
          pltpu.SemaphoreType.DMA,
          pltpu.SemaphoreType.DMA,
      ],
  )


def _bn(x, gamma, beta):
  mean = jnp.mean(x, axis=0, keepdims=True)
  var = jnp.mean((x - mean) ** 2, axis=0, keepdims=True)
  return (x - mean) * jax.lax.rsqrt(var + EPS) * gamma + beta


def _mlp_body(u_ref, i_ref, w1a_ref, w1b_ref, b1_ref, g1_ref, be1_ref,
              w2_ref, b2_ref, g2_ref, be2_ref, wout_ref, out_ref):
  x = (jnp.dot(u_ref[...], w1a_ref[...], preferred_element_type=jnp.float32)
       + jnp.dot(i_ref[...], w1b_ref[...], preferred_element_type=jnp.float32)
       + b1_ref[...])
  x = jnp.maximum(x, 0.0)
  x = _bn(x, g1_ref[...], be1_ref[...])
  x = jnp.dot(x, w2_ref[...], preferred_element_type=jnp.float32) + b2_ref[...]
  x = jnp.maximum(x, 0.0)
  x = _bn(x, g2_ref[...], be2_ref[...])
  logits = jnp.dot(x, wout_ref[...], preferred_element_type=jnp.float32)
  out_ref[...] = jax.nn.sigmoid(logits)


@jax.jit
def kernel(user_indices, item_indices, user_table, item_table,
           W1, b1, g1, be1, W2, b2, g2, be2, W_out):
  u_emb, i_emb = _make_sc_gather()(user_indices, item_indices,
                                   user_table, item_table)

  w1a = W1[:, :DIM].T
  w1b = W1[:, DIM:].T
  out = pl.pallas_call(
      _mlp_body,
      out_shape=jax.ShapeDtypeStruct((BATCH, 1), jnp.float32),
  )(u_emb, i_emb, w1a, w1b,
    b1.reshape(1, -1), g1.reshape(1, -1), be1.reshape(1, -1),
    W2.T, b2.reshape(1, -1), g2.reshape(1, -1), be2.reshape(1, -1),
    W_out.T)
  return out


# speedup vs baseline: 1.4001x; 1.4001x over previous
import functools

import jax
import jax.numpy as jnp
from jax import lax
from jax.experimental import pallas as pl
from jax.experimental.pallas import tpu as pltpu
from jax.experimental.pallas import tpu_sc as plsc

BATCH = 16384
DIM = 32
EPS = 1e-5

_CHUNK = 16


@functools.cache
def _make_sc_gather():
  info = plsc.get_sparse_core_info()
  nc, ns = info.num_cores, info.num_subcores
  nw = nc * ns
  b_per_w = BATCH // nw

  def _sc_gather_body(u_idx_hbm, i_idx_hbm, u_table_hbm, i_table_hbm,
                      u_out_hbm, i_out_hbm,
                      idx_u, idx_i, rows, sem):
    wid = lax.axis_index("s") * nc + lax.axis_index("c")
    base = wid * b_per_w
    pltpu.sync_copy(u_idx_hbm.at[pl.ds(base, b_per_w)], idx_u)
    pltpu.sync_copy(i_idx_hbm.at[pl.ds(base, b_per_w)], idx_i)

    def gather_one(table_hbm, idx_ref, out_hbm):
      def chunk_body(c, _):
        cbase = pl.multiple_of(c * _CHUNK, _CHUNK)
        iv = idx_ref[pl.ds(cbase, _CHUNK)]
        copies = []
        for k in range(_CHUNK):
          row = iv[k]
          copies.append(pltpu.async_copy(
              table_hbm.at[pl.ds(row, 1)],
              rows.at[pl.ds(cbase + k, 1)], sem))
        for cp in copies:
          cp.wait()
        return 0

      lax.fori_loop(0, b_per_w // _CHUNK, chunk_body, 0)
      pltpu.sync_copy(rows, out_hbm.at[pl.ds(base, b_per_w)])

    gather_one(u_table_hbm, idx_u, u_out_hbm)
    gather_one(i_table_hbm, idx_i, i_out_hbm)

  return pl.kernel(
      _sc_gather_body,
      mesh=plsc.VectorSubcoreMesh(core_axis_name="c", subcore_axis_name="s"),
      out_type=[
          jax.ShapeDtypeStruct((BATCH, DIM), jnp.float32),
          jax.ShapeDtypeStruct((BATCH, DIM), jnp.float32),
      ],
      scratch_types=[
          pltpu.VMEM((b_per_w,), jnp.int32),
          pltpu.VMEM((b_per_w,), jnp.int32),
          pltpu.VMEM((b_per_w, DIM), jnp.float32),
          pltpu.SemaphoreType.DMA,
      ],
  )


def _bn(x, gamma, beta):
  mean = jnp.mean(x, axis=0, keepdims=True)
  var = jnp.mean((x - mean) ** 2, axis=0, keepdims=True)
  return (x - mean) * jax.lax.rsqrt(var + EPS) * gamma + beta


def _mlp_body(u_ref, i_ref, w1a_ref, w1b_ref, b1_ref, g1_ref, be1_ref,
              w2_ref, b2_ref, g2_ref, be2_ref, wout_ref, out_ref):
  x = (jnp.dot(u_ref[...], w1a_ref[...], preferred_element_type=jnp.float32)
       + jnp.dot(i_ref[...], w1b_ref[...], preferred_element_type=jnp.float32)
       + b1_ref[...])
  x = jnp.maximum(x, 0.0)
  x = _bn(x, g1_ref[...], be1_ref[...])
  x = jnp.dot(x, w2_ref[...], preferred_element_type=jnp.float32) + b2_ref[...]
  x = jnp.maximum(x, 0.0)
  x = _bn(x, g2_ref[...], be2_ref[...])
  logits = jnp.dot(x, wout_ref[...], preferred_element_type=jnp.float32)
  out_ref[...] = jax.nn.sigmoid(logits)


@jax.jit
def kernel(user_indices, item_indices, user_table, item_table,
           W1, b1, g1, be1, W2, b2, g2, be2, W_out):
  u_emb, i_emb = _make_sc_gather()(user_indices, item_indices,
                                   user_table, item_table)

  w1a = W1[:, :DIM].T
  w1b = W1[:, DIM:].T
  out = pl.pallas_call(
      _mlp_body,
      out_shape=jax.ShapeDtypeStruct((BATCH, 1), jnp.float32),
  )(u_emb, i_emb, w1a, w1b,
    b1.reshape(1, -1), g1.reshape(1, -1), be1.reshape(1, -1),
    W2.T, b2.reshape(1, -1), g2.reshape(1, -1), be2.reshape(1, -1),
    W_out.T)
  return out
